# baseline (speedup 1.0000x reference)
import functools

import jax
import jax.numpy as jnp
from jax import lax
from jax.experimental import pallas as pl
from jax.experimental.pallas import tpu as pltpu
from jax.experimental.pallas import tpu_sc as plsc

N_NODES = 10000
HALF = 5000
ACC_ROWS = 5120
CHUNK = 128
N_CHUNKS = 2560
CHUNKS_PER_SUBCORE = N_CHUNKS // 16
EDGES_PER_SUBCORE = CHUNKS_PER_SUBCORE * CHUNK
D = 128
NBUF = 4


def _sc_agg_body(g_hbm, src_hbm, dst_hbm, z_hbm, out_hbm,
                 srcvs, dstvs, ldsts, rowss, acc, isems, dsems, gsems, ssems):
    c = lax.axis_index("c")
    s = lax.axis_index("s")
    base = pl.multiple_of(c * HALF, 8)
    trash = HALF + s

    acc_off = pl.multiple_of(s * (ACC_ROWS // 16), ACC_ROWS // 16)
    pltpu.sync_copy(z_hbm, acc.at[pl.ds(acc_off, ACC_ROWS // 16)])
    plsc.subcore_barrier()

    eoff = pl.multiple_of(s * EDGES_PER_SUBCORE, 128)

    def fire_ids(i, j):
        off = pl.multiple_of(eoff + i * CHUNK, 128)
        pltpu.async_copy(src_hbm.at[pl.ds(off, CHUNK)], srcvs[j], isems[j])
        pltpu.async_copy(dst_hbm.at[pl.ds(off, CHUNK)], dstvs[j], dsems[j])

    def compute_ldst(j):
        for k in range(CHUNK // 16):
            ld = dstvs[j][pl.ds(k * 16, 16)] - base
            ok = (ld >= 0) & (ld < HALF)
            ldsts[j][pl.ds(k * 16, 16)] = jnp.where(ok, ld, trash)

    def swait(j):
        pltpu.make_async_copy(rowss[j], acc.at[ldsts[j]], ssems[j]).wait()

    n_rounds = CHUNKS_PER_SUBCORE // NBUF

    def round_body(p, carry):
        for j in range(NBUF):
            def stage(j=j):
                swait(j)
            pl.when(p > 0)(stage)
            fire_ids(NBUF * p + j, j)
        for j in range(NBUF):
            pltpu.make_async_copy(
                src_hbm.at[pl.ds(eoff, CHUNK)], srcvs[j], isems[j]).wait()
            pltpu.make_async_copy(
                dst_hbm.at[pl.ds(eoff, CHUNK)], dstvs[j], dsems[j]).wait()
            compute_ldst(j)
            pltpu.async_copy(g_hbm.at[srcvs[j]], rowss[j], gsems[j])
        for j in range(NBUF):
            pltpu.make_async_copy(
                g_hbm.at[srcvs[j]], rowss[j], gsems[j]).wait()
            pltpu.async_copy(rowss[j], acc.at[ldsts[j]], ssems[j], add=True)
        return carry

    lax.fori_loop(0, n_rounds, round_body, 0)
    for j in range(NBUF):
        swait(j)
    plsc.subcore_barrier()

    @pl.when(s == 0)
    def _():
        pltpu.sync_copy(acc.at[pl.ds(0, HALF)], out_hbm.at[pl.ds(base, HALF)])


@functools.cache
def _sc_agg():
    mesh = plsc.VectorSubcoreMesh(core_axis_name="c", subcore_axis_name="s")
    return pl.kernel(
        _sc_agg_body,
        out_type=jax.ShapeDtypeStruct((N_NODES, D), jnp.float32),
        mesh=mesh,
        scratch_types=[
            [pltpu.VMEM((CHUNK,), jnp.int32)] * NBUF,
            [pltpu.VMEM((CHUNK,), jnp.int32)] * NBUF,
            [pltpu.VMEM((CHUNK,), jnp.int32)] * NBUF,
            [pltpu.VMEM((CHUNK, D), jnp.float32)] * NBUF,
            pltpu.VMEM_SHARED((ACC_ROWS, D), jnp.float32),
            [pltpu.SemaphoreType.DMA] * NBUF,
            [pltpu.SemaphoreType.DMA] * NBUF,
            [pltpu.SemaphoreType.DMA] * NBUF,
            [pltpu.SemaphoreType.DMA] * NBUF,
        ],
    )


def _mm_scale_body(x_ref, w_ref, deg_ref, oa_ref, ob_ref):
    dis = lax.rsqrt(deg_ref[...] + 1.0)
    o = jnp.dot(x_ref[...], w_ref[...],
                preferred_element_type=jnp.float32) * dis
    oa_ref[...] = o[:, :128]
    ob_ref[...] = o[:, 128:]


def _mm_scale(x, W, deg_col):
    M, K = x.shape
    _, N = W.shape
    BM = 2000
    return pl.pallas_call(
        _mm_scale_body,
        grid=(M // BM,),
        in_specs=[
            pl.BlockSpec((BM, K), lambda i: (i, 0)),
            pl.BlockSpec((K, N), lambda i: (0, 0)),
            pl.BlockSpec((BM, 1), lambda i: (i, 0)),
        ],
        out_specs=[pl.BlockSpec((BM, 128), lambda i: (i, 0)),
                   pl.BlockSpec((BM, 128), lambda i: (i, 0))],
        out_shape=[jax.ShapeDtypeStruct((M, 128), jnp.float32),
                   jax.ShapeDtypeStruct((M, 128), jnp.float32)],
    )(x, W, deg_col)


def _fused_body(split_out, aa_ref, ab_ref, ga_ref, gb_ref, deg_ref, b_ref,
                w_ref, *o_refs):
    dis = lax.rsqrt(deg_ref[...] + 1.0)
    agg = jnp.concatenate([aa_ref[...], ab_ref[...]], axis=1)
    g = jnp.concatenate([ga_ref[...], gb_ref[...]], axis=1)
    h = jnp.maximum((agg + g) * dis + b_ref[...], 0.0)
    o = jnp.dot(h, w_ref[...], preferred_element_type=jnp.float32) * dis
    if split_out:
        o_refs[0][...] = o[:, :128]
        o_refs[1][...] = o[:, 128:]
    else:
        o_refs[0][...] = o


def _fused(agg_a, agg_b, g_a, g_b, deg_col, b, W):
    M = g_a.shape[0]
    K = 2 * g_a.shape[1]
    _, N = W.shape
    BM = 2000
    split_out = N == 256
    n_out = 2 if split_out else 1
    return pl.pallas_call(
        functools.partial(_fused_body, split_out),
        grid=(M // BM,),
        in_specs=[
            pl.BlockSpec((BM, 128), lambda i: (i, 0)),
            pl.BlockSpec((BM, 128), lambda i: (i, 0)),
            pl.BlockSpec((BM, 128), lambda i: (i, 0)),
            pl.BlockSpec((BM, 128), lambda i: (i, 0)),
            pl.BlockSpec((BM, 1), lambda i: (i, 0)),
            pl.BlockSpec((1, K), lambda i: (0, 0)),
            pl.BlockSpec((K, N), lambda i: (0, 0)),
        ],
        out_specs=[pl.BlockSpec((BM, 128), lambda i: (i, 0))] * n_out,
        out_shape=[jax.ShapeDtypeStruct((M, 128), jnp.float32)] * n_out,
    )(agg_a, agg_b, g_a, g_b, deg_col, b.reshape(1, K), W)


def _final_body(agg_ref, g_ref, deg_ref, b_ref, o_ref):
    dis = lax.rsqrt(deg_ref[...] + 1.0)
    o_ref[...] = jnp.maximum(
        (agg_ref[...] + g_ref[...]) * dis + b_ref[...], 0.0)


def _final(agg, g, deg_col, b):
    M, K = g.shape
    BM = 2000
    return pl.pallas_call(
        _final_body,
        grid=(M // BM,),
        in_specs=[
            pl.BlockSpec((BM, K), lambda i: (i, 0)),
            pl.BlockSpec((BM, K), lambda i: (i, 0)),
            pl.BlockSpec((BM, 1), lambda i: (i, 0)),
            pl.BlockSpec((1, K), lambda i: (0, 0)),
        ],
        out_specs=pl.BlockSpec((BM, K), lambda i: (i, 0)),
        out_shape=jax.ShapeDtypeStruct((M, K), jnp.float32),
    )(agg, g, deg_col, b.reshape(1, K))


def kernel(x, edge_index, W1, b1, W2, b2, W3, b3):
    src = edge_index[0].astype(jnp.int32)
    dst = edge_index[1].astype(jnp.int32)
    pad = N_CHUNKS * CHUNK - src.shape[0]
    src1d = jnp.concatenate([src, jnp.zeros((pad,), jnp.int32)])
    dst1d = jnp.concatenate([dst, jnp.full((pad,), N_NODES, jnp.int32)])

    z = jnp.zeros((ACC_ROWS // 16, D), jnp.float32)
    ones = jnp.ones((N_NODES, D), jnp.float32)

    agg = _sc_agg()
    deg128 = agg(ones, src1d, dst1d, z)
    deg_col = deg128[:, :1]

    g1a, g1b = _mm_scale(x, W1, deg_col)
    agg1a = agg(g1a, src1d, dst1d, z)
    agg1b = agg(g1b, src1d, dst1d, z)
    g2a, g2b = _fused(agg1a, agg1b, g1a, g1b, deg_col, b1, W2)
    agg2a = agg(g2a, src1d, dst1d, z)
    agg2b = agg(g2b, src1d, dst1d, z)
    g3 = _fused(agg2a, agg2b, g2a, g2b, deg_col, b2, W3)[0]
    agg3 = agg(g3, src1d, dst1d, z)
    return _final(agg3, g3, deg_col, b3)


# speedup vs baseline: 2.8353x; 2.8353x over previous
import functools

import jax
import jax.numpy as jnp
from jax import lax
from jax.experimental import pallas as pl
from jax.experimental.pallas import tpu as pltpu
from jax.experimental.pallas import tpu_sc as plsc

N_NODES = 10000
HALF = 5000
ACC_ROWS = 5120
CHUNK = 128
N_CHUNKS = 2560
CHUNKS_PER_SUBCORE = N_CHUNKS // 16
EDGES_PER_SUBCORE = CHUNKS_PER_SUBCORE * CHUNK
D = 128
NBUF = 4


RING = 8


def _sc_agg_body(g_hbm, src_hbm, dst_hbm, z_hbm, out_hbm,
                 srcv1d, dstv1d, cbuf, csrcb, cldstb, rows, acc, sem):
    c = lax.axis_index("c")
    s = lax.axis_index("s")
    base = pl.multiple_of(c * HALF, 8)
    trash = HALF + s

    acc_off = pl.multiple_of(s * (ACC_ROWS // 16), ACC_ROWS // 16)
    pltpu.sync_copy(z_hbm, acc.at[pl.ds(acc_off, ACC_ROWS // 16)])

    eoff = pl.multiple_of(s * EDGES_PER_SUBCORE, 128)
    pltpu.sync_copy(src_hbm.at[pl.ds(eoff, EDGES_PER_SUBCORE)], srcv1d)
    pltpu.sync_copy(dst_hbm.at[pl.ds(eoff, EDGES_PER_SUBCORE)], dstv1d)
    plsc.subcore_barrier()

    zeros16 = jnp.zeros((16,), jnp.int32)
    iota16 = lax.iota(jnp.int32, 16)
    RB = RING * CHUNK

    def flush(q, nblk):
        qoff = pl.multiple_of(q * CHUNK, CHUNK)
        for k in range(CHUNK // 16):
            pk = cbuf[pl.ds(qoff + k * 16, 16)]
            csrcb[pl.ds(k * 16, 16)] = pk >> 13
            cldstb[pl.ds(k * 16, 16)] = pk & 8191
        pltpu.async_copy(g_hbm.at[csrcb], rows, sem).wait()
        pltpu.sync_copy(rows, acc.at[cldstb], add=True)

    def chunk_body(i, carry):
        wpos, rblk = carry
        for k in range(CHUNK // 16):
            off = pl.multiple_of(i * CHUNK + k * 16, 16)
            sv = srcv1d[pl.ds(off, 16)]
            ld = dstv1d[pl.ds(off, 16)] - base
            m = (ld >= 0) & (ld < HALF)
            key = jnp.where(m, iota16, 16 + iota16)
            packed = (sv << 13) | (ld & 8191)
            _, vs = lax.sort((key, packed), num_keys=1)
            o = wpos & (RB - 1)
            cbuf[pl.ds(o, 16)] = vs

            def spill_fix():
                cbuf[pl.ds(0, 16)] = cbuf[pl.ds(RB, 16)]

            pl.when(o > RB - 16)(spill_fix)
            cntv = plsc.all_reduce_population_count(m)
            wpos = wpos + cntv[0]

        def do_flush():
            flush(rblk & (RING - 1), rblk)

        full = wpos - rblk * CHUNK >= CHUNK
        pl.when(full)(do_flush)
        return wpos, jnp.where(full, rblk + 1, rblk)

    wpos, rblk = lax.fori_loop(0, CHUNKS_PER_SUBCORE, chunk_body, (0, 0))

    trash_packed = jnp.full((16,), trash, jnp.int32)
    for k in range(CHUNK // 16):
        o = wpos & (RB - 1)
        cbuf[pl.ds(o, 16)] = trash_packed

        def spill_fix2():
            cbuf[pl.ds(0, 16)] = cbuf[pl.ds(RB, 16)]

        pl.when(o > RB - 16)(spill_fix2)
        wpos = wpos + 16

    def tail_flush():
        flush(rblk & (RING - 1), rblk)

    pl.when(wpos - 128 > rblk * CHUNK - CHUNK)(tail_flush)
    plsc.subcore_barrier()

    @pl.when(s == 0)
    def _():
        pltpu.sync_copy(acc.at[pl.ds(0, HALF)], out_hbm.at[pl.ds(base, HALF)])


@functools.cache
def _sc_agg():
    mesh = plsc.VectorSubcoreMesh(core_axis_name="c", subcore_axis_name="s")
    return pl.kernel(
        _sc_agg_body,
        out_type=jax.ShapeDtypeStruct((N_NODES, D), jnp.float32),
        mesh=mesh,
        compiler_params=pltpu.CompilerParams(needs_layout_passes=False),
        scratch_types=[
            pltpu.VMEM((EDGES_PER_SUBCORE,), jnp.int32),
            pltpu.VMEM((EDGES_PER_SUBCORE,), jnp.int32),
            pltpu.VMEM((RING * CHUNK + 16,), jnp.int32),
            pltpu.VMEM((CHUNK,), jnp.int32),
            pltpu.VMEM((CHUNK,), jnp.int32),
            pltpu.VMEM((CHUNK, D), jnp.float32),
            pltpu.VMEM_SHARED((ACC_ROWS, D), jnp.float32),
            pltpu.SemaphoreType.DMA,
        ],
    )


def _mm_scale_body(x_ref, w_ref, deg_ref, oa_ref, ob_ref):
    dis = lax.rsqrt(deg_ref[...] + 1.0)
    o = jnp.dot(x_ref[...], w_ref[...],
                preferred_element_type=jnp.float32) * dis
    oa_ref[...] = o[:, :128]
    ob_ref[...] = o[:, 128:]


def _mm_scale(x, W, deg_col):
    M, K = x.shape
    _, N = W.shape
    BM = 2000
    return pl.pallas_call(
        _mm_scale_body,
        grid=(M // BM,),
        in_specs=[
            pl.BlockSpec((BM, K), lambda i: (i, 0)),
            pl.BlockSpec((K, N), lambda i: (0, 0)),
            pl.BlockSpec((BM, 1), lambda i: (i, 0)),
        ],
        out_specs=[pl.BlockSpec((BM, 128), lambda i: (i, 0)),
                   pl.BlockSpec((BM, 128), lambda i: (i, 0))],
        out_shape=[jax.ShapeDtypeStruct((M, 128), jnp.float32),
                   jax.ShapeDtypeStruct((M, 128), jnp.float32)],
    )(x, W, deg_col)


def _fused_body(split_out, aa_ref, ab_ref, ga_ref, gb_ref, deg_ref, b_ref,
                w_ref, *o_refs):
    dis = lax.rsqrt(deg_ref[...] + 1.0)
    agg = jnp.concatenate([aa_ref[...], ab_ref[...]], axis=1)
    g = jnp.concatenate([ga_ref[...], gb_ref[...]], axis=1)
    h = jnp.maximum((agg + g) * dis + b_ref[...], 0.0)
    o = jnp.dot(h, w_ref[...], preferred_element_type=jnp.float32) * dis
    if split_out:
        o_refs[0][...] = o[:, :128]
        o_refs[1][...] = o[:, 128:]
    else:
        o_refs[0][...] = o


def _fused(agg_a, agg_b, g_a, g_b, deg_col, b, W):
    M = g_a.shape[0]
    K = 2 * g_a.shape[1]
    _, N = W.shape
    BM = 2000
    split_out = N == 256
    n_out = 2 if split_out else 1
    return pl.pallas_call(
        functools.partial(_fused_body, split_out),
        grid=(M // BM,),
        in_specs=[
            pl.BlockSpec((BM, 128), lambda i: (i, 0)),
            pl.BlockSpec((BM, 128), lambda i: (i, 0)),
            pl.BlockSpec((BM, 128), lambda i: (i, 0)),
            pl.BlockSpec((BM, 128), lambda i: (i, 0)),
            pl.BlockSpec((BM, 1), lambda i: (i, 0)),
            pl.BlockSpec((1, K), lambda i: (0, 0)),
            pl.BlockSpec((K, N), lambda i: (0, 0)),
        ],
        out_specs=[pl.BlockSpec((BM, 128), lambda i: (i, 0))] * n_out,
        out_shape=[jax.ShapeDtypeStruct((M, 128), jnp.float32)] * n_out,
    )(agg_a, agg_b, g_a, g_b, deg_col, b.reshape(1, K), W)


def _final_body(agg_ref, g_ref, deg_ref, b_ref, o_ref):
    dis = lax.rsqrt(deg_ref[...] + 1.0)
    o_ref[...] = jnp.maximum(
        (agg_ref[...] + g_ref[...]) * dis + b_ref[...], 0.0)


def _final(agg, g, deg_col, b):
    M, K = g.shape
    BM = 2000
    return pl.pallas_call(
        _final_body,
        grid=(M // BM,),
        in_specs=[
            pl.BlockSpec((BM, K), lambda i: (i, 0)),
            pl.BlockSpec((BM, K), lambda i: (i, 0)),
            pl.BlockSpec((BM, 1), lambda i: (i, 0)),
            pl.BlockSpec((1, K), lambda i: (0, 0)),
        ],
        out_specs=pl.BlockSpec((BM, K), lambda i: (i, 0)),
        out_shape=jax.ShapeDtypeStruct((M, K), jnp.float32),
    )(agg, g, deg_col, b.reshape(1, K))


def kernel(x, edge_index, W1, b1, W2, b2, W3, b3):
    src = edge_index[0].astype(jnp.int32)
    dst = edge_index[1].astype(jnp.int32)
    pad = N_CHUNKS * CHUNK - src.shape[0]
    src1d = jnp.concatenate([src, jnp.zeros((pad,), jnp.int32)])
    dst1d = jnp.concatenate([dst, jnp.full((pad,), N_NODES, jnp.int32)])

    z = jnp.zeros((ACC_ROWS // 16, D), jnp.float32)
    ones = jnp.ones((N_NODES, D), jnp.float32)

    agg = _sc_agg()
    deg128 = agg(ones, src1d, dst1d, z)
    deg_col = deg128[:, :1]

    g1a, g1b = _mm_scale(x, W1, deg_col)
    agg1a = agg(g1a, src1d, dst1d, z)
    agg1b = agg(g1b, src1d, dst1d, z)
    g2a, g2b = _fused(agg1a, agg1b, g1a, g1b, deg_col, b1, W2)
    agg2a = agg(g2a, src1d, dst1d, z)
    agg2b = agg(g2b, src1d, dst1d, z)
    g3 = _fused(agg2a, agg2b, g2a, g2b, deg_col, b2, W3)[0]
    agg3 = agg(g3, src1d, dst1d, z)
    return _final(agg3, g3, deg_col, b3)


# speedup vs baseline: 3.3725x; 1.1895x over previous
import functools

import jax
import jax.numpy as jnp
from jax import lax
from jax.experimental import pallas as pl
from jax.experimental.pallas import tpu as pltpu
from jax.experimental.pallas import tpu_sc as plsc

N_NODES = 10000
HALF = 5000
ACC_ROWS = 5120
CHUNK = 128
N_CHUNKS = 2560
CHUNKS_PER_SUBCORE = N_CHUNKS // 16
EDGES_PER_SUBCORE = CHUNKS_PER_SUBCORE * CHUNK
D = 128
NBUF = 4


RING = 8


def _sc_agg_body(g_hbm, src_hbm, dst_hbm, z_hbm, out_hbm,
                 srcv1d, dstv1d, cbuf, csrcb, cldstb, rows,
                 csrcb1, cldstb1, rows1, acc, gsem, sem, sem1):
    c = lax.axis_index("c")
    s = lax.axis_index("s")
    base = pl.multiple_of(c * HALF, 8)
    trash = HALF + s

    acc_off = pl.multiple_of(s * (ACC_ROWS // 16), ACC_ROWS // 16)
    pltpu.sync_copy(z_hbm, acc.at[pl.ds(acc_off, ACC_ROWS // 16)])

    eoff = pl.multiple_of(s * EDGES_PER_SUBCORE, 128)
    pltpu.sync_copy(src_hbm.at[pl.ds(eoff, EDGES_PER_SUBCORE)], srcv1d)
    pltpu.sync_copy(dst_hbm.at[pl.ds(eoff, EDGES_PER_SUBCORE)], dstv1d)
    plsc.subcore_barrier()

    zeros16 = jnp.zeros((16,), jnp.int32)
    iota16 = lax.iota(jnp.int32, 16)
    RB = RING * CHUNK

    def flush_p(nblk, rows, csrcb, cldstb, ssem):
        def drain():
            pltpu.make_async_copy(rows, acc.at[cldstb], ssem).wait()

        pl.when(nblk >= 2)(drain)
        qoff = pl.multiple_of((nblk & (RING - 1)) * CHUNK, CHUNK)
        for k in range(CHUNK // 16):
            pk = cbuf[pl.ds(qoff + k * 16, 16)]
            csrcb[pl.ds(k * 16, 16)] = pk >> 13
            cldstb[pl.ds(k * 16, 16)] = pk & 8191
        pltpu.async_copy(g_hbm.at[csrcb], rows, gsem).wait()
        pltpu.async_copy(rows, acc.at[cldstb], ssem, add=True)

    def flush(q, nblk):
        def even():
            flush_p(nblk, rows, csrcb, cldstb, sem)

        def odd():
            flush_p(nblk, rows1, csrcb1, cldstb1, sem1)

        pl.when((nblk & 1) == 0)(even)
        pl.when((nblk & 1) == 1)(odd)

    def chunk_body(i, carry):
        wpos, rblk = carry
        for k in range(CHUNK // 16):
            off = pl.multiple_of(i * CHUNK + k * 16, 16)
            sv = srcv1d[pl.ds(off, 16)]
            ld = dstv1d[pl.ds(off, 16)] - base
            m = (ld >= 0) & (ld < HALF)
            key = jnp.where(m, iota16, 16 + iota16)
            packed = (sv << 13) | (ld & 8191)
            _, vs = lax.sort((key, packed), num_keys=1)
            o = wpos & (RB - 1)
            cbuf[pl.ds(o, 16)] = vs

            def spill_fix():
                cbuf[pl.ds(0, 16)] = cbuf[pl.ds(RB, 16)]

            pl.when(o > RB - 16)(spill_fix)
            cntv = plsc.all_reduce_population_count(m)
            wpos = wpos + cntv[0]

        def do_flush():
            flush(rblk & (RING - 1), rblk)

        full = wpos - rblk * CHUNK >= CHUNK
        pl.when(full)(do_flush)
        return wpos, jnp.where(full, rblk + 1, rblk)

    wpos, rblk = lax.fori_loop(0, CHUNKS_PER_SUBCORE, chunk_body, (0, 0))

    trash_packed = jnp.full((16,), trash, jnp.int32)
    for k in range(CHUNK // 16):
        o = wpos & (RB - 1)
        cbuf[pl.ds(o, 16)] = trash_packed

        def spill_fix2():
            cbuf[pl.ds(0, 16)] = cbuf[pl.ds(RB, 16)]

        pl.when(o > RB - 16)(spill_fix2)
        wpos = wpos + 16

    def tail_flush():
        flush(rblk & (RING - 1), rblk)

    tail = wpos - 128 > rblk * CHUNK - CHUNK
    pl.when(tail)(tail_flush)
    nf = jnp.where(tail, rblk + 1, rblk)

    def drain_even():
        pltpu.make_async_copy(rows, acc.at[cldstb], sem).wait()

    def drain_odd():
        pltpu.make_async_copy(rows1, acc.at[cldstb1], sem1).wait()

    pl.when(nf >= 1)(drain_even)
    pl.when(nf >= 2)(drain_odd)
    plsc.subcore_barrier()

    @pl.when(s == 0)
    def _():
        pltpu.sync_copy(acc.at[pl.ds(0, HALF)], out_hbm.at[pl.ds(base, HALF)])


@functools.cache
def _sc_agg():
    mesh = plsc.VectorSubcoreMesh(core_axis_name="c", subcore_axis_name="s")
    return pl.kernel(
        _sc_agg_body,
        out_type=jax.ShapeDtypeStruct((N_NODES, D), jnp.float32),
        mesh=mesh,
        compiler_params=pltpu.CompilerParams(needs_layout_passes=False),
        scratch_types=[
            pltpu.VMEM((EDGES_PER_SUBCORE,), jnp.int32),
            pltpu.VMEM((EDGES_PER_SUBCORE,), jnp.int32),
            pltpu.VMEM((RING * CHUNK + 16,), jnp.int32),
            pltpu.VMEM((CHUNK,), jnp.int32),
            pltpu.VMEM((CHUNK,), jnp.int32),
            pltpu.VMEM((CHUNK, D), jnp.float32),
            pltpu.VMEM((CHUNK,), jnp.int32),
            pltpu.VMEM((CHUNK,), jnp.int32),
            pltpu.VMEM((CHUNK, D), jnp.float32),
            pltpu.VMEM_SHARED((ACC_ROWS, D), jnp.float32),
            pltpu.SemaphoreType.DMA,
            pltpu.SemaphoreType.DMA,
            pltpu.SemaphoreType.DMA,
        ],
    )


def _mm_scale_body(x_ref, w_ref, deg_ref, oa_ref, ob_ref):
    dis = lax.rsqrt(deg_ref[...] + 1.0)
    o = jnp.dot(x_ref[...], w_ref[...],
                preferred_element_type=jnp.float32) * dis
    oa_ref[...] = o[:, :128]
    ob_ref[...] = o[:, 128:]


def _mm_scale(x, W, deg_col):
    M, K = x.shape
    _, N = W.shape
    BM = 2000
    return pl.pallas_call(
        _mm_scale_body,
        grid=(M // BM,),
        in_specs=[
            pl.BlockSpec((BM, K), lambda i: (i, 0)),
            pl.BlockSpec((K, N), lambda i: (0, 0)),
            pl.BlockSpec((BM, 1), lambda i: (i, 0)),
        ],
        out_specs=[pl.BlockSpec((BM, 128), lambda i: (i, 0)),
                   pl.BlockSpec((BM, 128), lambda i: (i, 0))],
        out_shape=[jax.ShapeDtypeStruct((M, 128), jnp.float32),
                   jax.ShapeDtypeStruct((M, 128), jnp.float32)],
    )(x, W, deg_col)


def _fused_body(split_out, aa_ref, ab_ref, ga_ref, gb_ref, deg_ref, b_ref,
                w_ref, *o_refs):
    dis = lax.rsqrt(deg_ref[...] + 1.0)
    agg = jnp.concatenate([aa_ref[...], ab_ref[...]], axis=1)
    g = jnp.concatenate([ga_ref[...], gb_ref[...]], axis=1)
    h = jnp.maximum((agg + g) * dis + b_ref[...], 0.0)
    o = jnp.dot(h, w_ref[...], preferred_element_type=jnp.float32) * dis
    if split_out:
        o_refs[0][...] = o[:, :128]
        o_refs[1][...] = o[:, 128:]
    else:
        o_refs[0][...] = o


def _fused(agg_a, agg_b, g_a, g_b, deg_col, b, W):
    M = g_a.shape[0]
    K = 2 * g_a.shape[1]
    _, N = W.shape
    BM = 2000
    split_out = N == 256
    n_out = 2 if split_out else 1
    return pl.pallas_call(
        functools.partial(_fused_body, split_out),
        grid=(M // BM,),
        in_specs=[
            pl.BlockSpec((BM, 128), lambda i: (i, 0)),
            pl.BlockSpec((BM, 128), lambda i: (i, 0)),
            pl.BlockSpec((BM, 128), lambda i: (i, 0)),
            pl.BlockSpec((BM, 128), lambda i: (i, 0)),
            pl.BlockSpec((BM, 1), lambda i: (i, 0)),
            pl.BlockSpec((1, K), lambda i: (0, 0)),
            pl.BlockSpec((K, N), lambda i: (0, 0)),
        ],
        out_specs=[pl.BlockSpec((BM, 128), lambda i: (i, 0))] * n_out,
        out_shape=[jax.ShapeDtypeStruct((M, 128), jnp.float32)] * n_out,
    )(agg_a, agg_b, g_a, g_b, deg_col, b.reshape(1, K), W)


def _final_body(agg_ref, g_ref, deg_ref, b_ref, o_ref):
    dis = lax.rsqrt(deg_ref[...] + 1.0)
    o_ref[...] = jnp.maximum(
        (agg_ref[...] + g_ref[...]) * dis + b_ref[...], 0.0)


def _final(agg, g, deg_col, b):
    M, K = g.shape
    BM = 2000
    return pl.pallas_call(
        _final_body,
        grid=(M // BM,),
        in_specs=[
            pl.BlockSpec((BM, K), lambda i: (i, 0)),
            pl.BlockSpec((BM, K), lambda i: (i, 0)),
            pl.BlockSpec((BM, 1), lambda i: (i, 0)),
            pl.BlockSpec((1, K), lambda i: (0, 0)),
        ],
        out_specs=pl.BlockSpec((BM, K), lambda i: (i, 0)),
        out_shape=jax.ShapeDtypeStruct((M, K), jnp.float32),
    )(agg, g, deg_col, b.reshape(1, K))


def kernel(x, edge_index, W1, b1, W2, b2, W3, b3):
    src = edge_index[0].astype(jnp.int32)
    dst = edge_index[1].astype(jnp.int32)
    pad = N_CHUNKS * CHUNK - src.shape[0]
    src1d = jnp.concatenate([src, jnp.zeros((pad,), jnp.int32)])
    dst1d = jnp.concatenate([dst, jnp.full((pad,), N_NODES, jnp.int32)])

    z = jnp.zeros((ACC_ROWS // 16, D), jnp.float32)
    ones = jnp.ones((N_NODES, D), jnp.float32)

    agg = _sc_agg()
    deg128 = agg(ones, src1d, dst1d, z)
    deg_col = deg128[:, :1]

    g1a, g1b = _mm_scale(x, W1, deg_col)
    agg1a = agg(g1a, src1d, dst1d, z)
    agg1b = agg(g1b, src1d, dst1d, z)
    g2a, g2b = _fused(agg1a, agg1b, g1a, g1b, deg_col, b1, W2)
    agg2a = agg(g2a, src1d, dst1d, z)
    agg2b = agg(g2b, src1d, dst1d, z)
    g3 = _fused(agg2a, agg2b, g2a, g2b, deg_col, b2, W3)[0]
    agg3 = agg(g3, src1d, dst1d, z)
    return _final(agg3, g3, deg_col, b3)
